# baseline (speedup 1.0000x reference)
import functools

import jax
import jax.numpy as jnp
from jax import lax
from jax.experimental import pallas as pl
from jax.experimental.pallas import tpu as pltpu
from jax.experimental.pallas import tpu_sc as plsc

B = 16384
D = 32
LANES = 128

_info = plsc.get_sparse_core_info()
NC = _info.num_cores
NS = _info.num_subcores
L = _info.num_lanes
NW = NC * NS
BPW = B // NW

WPT = 245
SPAN = 5 * LANES
NCHK = 49
MAXOFF = 999424
DUMP = B
WLCAP = 1072
STCAP = 272
NVR = B // L


def _sc_extract(dflat, ut, it):
    mesh = plsc.VectorSubcoreMesh(core_axis_name="c", subcore_axis_name="s")
    stage_t = jax.ShapeDtypeStruct((B + L, LANES), jnp.float32)

    @functools.partial(
        pl.kernel,
        mesh=mesh,
        compiler_params=pltpu.CompilerParams(
            needs_layout_passes=False, use_tc_tiling_on_sc=True),
        out_type=(stage_t, stage_t),
        scratch_types=[
            pltpu.VMEM((2 * B,), jnp.int32),
            pltpu.VMEM((NVR,), jnp.int32),
            pltpu.VMEM((NVR,), jnp.int32),
            pltpu.VMEM((NVR,), jnp.int32),
            pltpu.VMEM((NVR,), jnp.int32),
            pltpu.VMEM((WLCAP,), jnp.int32),
            pltpu.VMEM((WLCAP,), jnp.int32),
            pltpu.VMEM((WLCAP,), jnp.int32),
            pltpu.VMEM((WLCAP,), jnp.int32),
            pltpu.VMEM((80,), jnp.int32),
            pltpu.VMEM((80,), jnp.int32),
            pltpu.VMEM((STCAP,), jnp.int32),
            pltpu.VMEM((STCAP,), jnp.int32),
            pltpu.VMEM((2, D, SPAN), jnp.float32),
            pltpu.VMEM((4, L, LANES), jnp.float32),
            pltpu.VMEM((4, L), jnp.int32),
            pltpu.SemaphoreType.DMA,
            pltpu.SemaphoreType.DMA,
            pltpu.SemaphoreType.DMA,
        ],
    )
    def k(d_hbm, ut_hbm, it_hbm, u_out, v_out,
          data_vm, pc_u, pc_v, ba_u, ba_v,
          wlu_i, wlu_b, wlv_i, wlv_b, pc2, ba2, st_l, st_b,
          ring, st_src, b2d, semf0, semf1, ssem):
        wid = lax.axis_index("s") * NC + lax.axis_index("c")
        lo = wid * (WPT * LANES)
        hi = lo + WPT * LANES

        pltpu.sync_copy(d_hbm, data_vm)

        iota = lax.iota(jnp.int32, L)

        def ph1(g, carry):
            p0 = pl.multiple_of(g * L, L)
            two = 2 * p0 + 2 * iota
            uu = plsc.load_gather(data_vm, [two])
            vv = plsc.load_gather(data_vm, [two + 1])
            mu = (uu >= lo) & (uu < hi)
            mv = (vv >= lo) & (vv < hi)
            gv = jnp.full((L,), g, jnp.int32)
            plsc.store_scatter(
                pc_u, [gv], plsc.all_reduce_population_count(mu))
            plsc.store_scatter(
                pc_v, [gv], plsc.all_reduce_population_count(mv))
            return carry

        lax.fori_loop(0, NVR, ph1, 0)

        def prefix(pc, ba, nv):
            def pf(g, carry):
                g0 = pl.multiple_of(g * L, L)
                vec = pc[pl.ds(g0, L)]
                inc = plsc.cumsum(vec)
                ba[pl.ds(g0, L)] = carry + inc - vec
                return carry + jnp.sum(vec)

            return lax.fori_loop(0, nv, pf, jnp.int32(0))

        cnt_u = prefix(pc_u, ba_u, NVR // L)
        cnt_v = prefix(pc_v, ba_v, NVR // L)

        def ph3(g, carry):
            p0 = pl.multiple_of(g * L, L)
            two = 2 * p0 + 2 * iota
            gv = jnp.full((L,), g, jnp.int32)
            bpos = p0 + iota
            for (wl_i, wl_b, ba, col) in ((wlu_i, wlu_b, ba_u, 0),
                                          (wlv_i, wlv_b, ba_v, 1)):
                xx = plsc.load_gather(data_vm, [two + col])
                m = (xx >= lo) & (xx < hi)
                mi = m.astype(jnp.int32)
                rank = plsc.cumsum(mi) - mi
                pos = plsc.load_gather(ba, [gv]) + rank
                plsc.store_scatter(wl_i, [pos], xx, mask=m)
                plsc.store_scatter(wl_b, [pos], bpos, mask=m)
            return carry

        lax.fori_loop(0, NVR, ph3, 0)

        wlu_i[pl.ds(cnt_u, L)] = jnp.full((L,), -1, jnp.int32)
        wlu_b[pl.ds(cnt_u, L)] = jnp.full((L,), DUMP, jnp.int32)
        wlv_i[pl.ds(cnt_v, L)] = jnp.full((L,), -1, jnp.int32)
        wlv_b[pl.ds(cnt_v, L)] = jnp.full((L,), DUMP, jnp.int32)

        for wl_i, wl_b, cnt, tbl_hbm, stg in (
                (wlu_i, wlu_b, cnt_u, ut_hbm, u_out),
                (wlv_i, wlv_b, cnt_v, it_hbm, v_out)):
            nwl = (cnt + L - 1) // L

            for z in range(80 // L):
                pc2[pl.ds(z * L, L)] = jnp.zeros((L,), jnp.int32)

            def fetch(c, slot, sem, tbl_hbm=tbl_hbm):
                base = lo + c * SPAN
                off = pl.multiple_of(jnp.minimum(base, MAXOFF), LANES)
                pltpu.async_copy(
                    tbl_hbm.at[:, pl.ds(off, SPAN)], ring.at[slot], sem)

            def drainf(slot, sem, tbl_hbm=tbl_hbm):
                pltpu.make_async_copy(
                    tbl_hbm.at[:, pl.ds(0, SPAN)], ring.at[slot],
                    sem).wait()

            def process(c, slot, rc, wl_i=wl_i, wl_b=wl_b, nwl=nwl,
                        stg=stg):
                base = lo + c * SPAN
                off = jnp.minimum(base, MAXOFF)

                def c1(j, carry):
                    j0 = pl.multiple_of(j * L, L)
                    wi = wl_i[pl.ds(j0, L)]
                    m2 = (wi >= base) & (wi < base + SPAN)
                    plsc.store_scatter(
                        pc2, [jnp.full((L,), j, jnp.int32)],
                        plsc.all_reduce_population_count(m2))
                    return carry

                lax.fori_loop(0, nwl, c1, 0)

                def c2(g, carry):
                    g0 = pl.multiple_of(g * L, L)
                    vec = pc2[pl.ds(g0, L)]
                    inc = plsc.cumsum(vec)
                    ba2[pl.ds(g0, L)] = carry + inc - vec
                    return carry + jnp.sum(vec)

                cnt2 = lax.fori_loop(0, (nwl + L - 1) // L, c2,
                                     jnp.int32(0))

                def c3(j, carry):
                    j0 = pl.multiple_of(j * L, L)
                    wi = wl_i[pl.ds(j0, L)]
                    wb = wl_b[pl.ds(j0, L)]
                    m2 = (wi >= base) & (wi < base + SPAN)
                    mi = m2.astype(jnp.int32)
                    rank = plsc.cumsum(mi) - mi
                    pos = plsc.load_gather(
                        ba2, [jnp.full((L,), j, jnp.int32)]) + rank
                    plsc.store_scatter(st_l, [pos], wi - off, mask=m2)
                    plsc.store_scatter(st_b, [pos], wb, mask=m2)
                    return carry

                lax.fori_loop(0, nwl, c3, 0)

                st_l[pl.ds(cnt2, L)] = jnp.zeros((L,), jnp.int32)
                st_b[pl.ds(cnt2, L)] = jnp.full((L,), DUMP, jnp.int32)

                def group(gi, rc_in):
                    g0 = pl.multiple_of(gi * L, L)
                    lane_v = jnp.clip(st_l[pl.ds(g0, L)], 0, SPAN - 1)
                    b_v = jnp.clip(st_b[pl.ds(g0, L)], 0, DUMP)
                    slot4 = lax.rem(rc_in, 4)

                    @pl.when(rc_in >= 4)
                    def _():
                        pltpu.make_async_copy(
                            st_src.at[0], stg.at[b2d.at[0]], ssem).wait()

                    b2d[slot4, pl.ds(0, L)] = b_v
                    s4v = jnp.full((L,), slot4, jnp.int32)
                    for j in range(D):
                        jv = jnp.full((L,), j, jnp.int32)
                        vals = plsc.load_gather(
                            ring.at[slot], [jv, lane_v])
                        plsc.store_scatter(st_src, [s4v, iota, jv], vals)
                    pltpu.async_copy(
                        st_src.at[slot4], stg.at[b2d.at[slot4]], ssem)
                    return rc_in + 1

                return lax.fori_loop(0, (cnt2 + L - 1) // L, group, rc)

            fetch(0, 0, semf0)

            def blk(cb, rc):
                c0 = 2 * cb
                fetch(c0 + 1, 1, semf1)
                drainf(0, semf0)
                rc = process(c0, 0, rc)
                fetch(c0 + 2, 0, semf0)
                drainf(1, semf1)
                rc = process(c0 + 1, 1, rc)
                return rc

            rc = lax.fori_loop(0, (NCHK - 1) // 2, blk, jnp.int32(0))
            drainf(0, semf0)
            rc = process(NCHK - 1, 0, rc)

            def sdrain(i, carry):
                pltpu.make_async_copy(
                    st_src.at[0], stg.at[b2d.at[0]], ssem).wait()
                return carry

            lax.fori_loop(0, jnp.minimum(rc, 4), sdrain, 0)

    return k(dflat, ut, it)


def _sc_join(u_stage, v_stage):
    mesh = plsc.VectorSubcoreMesh(core_axis_name="c", subcore_axis_name="s")
    SUB = 128

    @functools.partial(
        pl.kernel,
        mesh=mesh,
        compiler_params=pltpu.CompilerParams(
            needs_layout_passes=False, use_tc_tiling_on_sc=True),
        out_type=jax.ShapeDtypeStruct((B,), jnp.float32),
        scratch_types=[
            pltpu.VMEM((SUB, LANES), jnp.float32),
            pltpu.VMEM((SUB, LANES), jnp.float32),
            pltpu.VMEM((BPW,), jnp.float32),
        ],
    )
    def k(u_hbm, v_hbm, out_hbm, ub, vb, out_v):
        wid = lax.axis_index("s") * NC + lax.axis_index("c")
        iota = lax.iota(jnp.int32, L)
        for sub in range(BPW // SUB):
            row0 = pl.multiple_of(wid * BPW + sub * SUB, 8)
            pltpu.sync_copy(u_hbm.at[pl.ds(row0, SUB)], ub)
            pltpu.sync_copy(v_hbm.at[pl.ds(row0, SUB)], vb)

            def dot16(g, carry, sub=sub):
                rl = g * L + iota
                acc = jnp.zeros((L,), jnp.float32)
                for j in range(D):
                    jv = jnp.full((L,), j, jnp.int32)
                    acc = acc + (plsc.load_gather(ub, [rl, jv])
                                 * plsc.load_gather(vb, [rl, jv]))
                o0 = pl.multiple_of(sub * SUB + g * L, L)
                out_v[pl.ds(o0, L)] = acc
                return carry

            lax.fori_loop(0, SUB // L, dot16, 0)

        base = pl.multiple_of(wid * BPW, BPW)
        pltpu.sync_copy(out_v, out_hbm.at[pl.ds(base, BPW)])

    return k(u_stage, v_stage)


def kernel(data, user_factors, item_factors):
    dflat = data.astype(jnp.int32).reshape(-1)
    u_stage, v_stage = _sc_extract(
        dflat, user_factors.T, item_factors.T)
    return _sc_join(u_stage, v_stage)


# speedup vs baseline: 5.1449x; 5.1449x over previous
import functools

import jax
import jax.numpy as jnp
from jax import lax
from jax.experimental import pallas as pl
from jax.experimental.pallas import tpu as pltpu
from jax.experimental.pallas import tpu_sc as plsc

B = 16384
D = 32
LANES = 128

_info = plsc.get_sparse_core_info()
NC = _info.num_cores
NS = _info.num_subcores
L = _info.num_lanes
NW = NC * NS
BPW = B // NW
NB = 8


def _sc_dot(data2, ut, it):
    mesh = plsc.VectorSubcoreMesh(core_axis_name="c", subcore_axis_name="s")

    @functools.partial(
        pl.kernel,
        mesh=mesh,
        compiler_params=pltpu.CompilerParams(
            needs_layout_passes=False, use_tc_tiling_on_sc=True),
        out_type=jax.ShapeDtypeStruct((B,), jnp.float32),
        scratch_types=[
            pltpu.VMEM((2 * BPW,), jnp.int32),
            pltpu.VMEM((NB, D, LANES), jnp.float32),
            pltpu.VMEM((NB, D, LANES), jnp.float32),
            pltpu.VMEM((BPW * L,), jnp.float32),
            pltpu.VMEM((BPW,), jnp.float32),
            [pltpu.SemaphoreType.DMA] * NB,
        ],
    )
    def k(d2_hbm, ut_hbm, it_hbm, out_hbm,
          idx_vm, win_u, win_v, part, out_v, sems):
        wid = lax.axis_index("s") * NC + lax.axis_index("c")

        pltpu.sync_copy(d2_hbm.at[wid], idx_vm)

        lanes = lax.iota(jnp.int32, L)

        def scal2(r):
            pos = jnp.minimum(2 * r, 2 * BPW - 2)
            base = pl.multiple_of((pos // L) * L, L)
            vec = idx_vm[pl.ds(base, L)]
            off = pos - base
            zero = jnp.zeros((L,), jnp.int32)
            iu = jnp.sum(jnp.where(lanes == jnp.full((L,), off), vec, zero))
            ii = jnp.sum(
                jnp.where(lanes == jnp.full((L,), off + 1), vec, zero))
            return iu, ii

        def fetch(slot, sem, iu, ii):
            cu = pl.multiple_of((iu // LANES) * LANES, LANES)
            ci = pl.multiple_of((ii // LANES) * LANES, LANES)
            pltpu.async_copy(
                ut_hbm.at[:, pl.ds(cu, LANES)], win_u.at[slot], sem)
            pltpu.async_copy(
                it_hbm.at[:, pl.ds(ci, LANES)], win_v.at[slot], sem)

        def drain(slot, sem):
            pltpu.make_async_copy(
                ut_hbm.at[:, pl.ds(0, LANES)], win_u.at[slot], sem).wait()
            pltpu.make_async_copy(
                it_hbm.at[:, pl.ds(0, LANES)], win_v.at[slot], sem).wait()

        rows_hi = lanes + L

        def compute(r, slot, iu, ii):
            lane_u = jnp.full((L,), iu % LANES, jnp.int32)
            lane_i = jnp.full((L,), ii % LANES, jnp.int32)
            u_lo = plsc.load_gather(win_u.at[slot], [lanes, lane_u])
            u_hi = plsc.load_gather(win_u.at[slot], [rows_hi, lane_u])
            v_lo = plsc.load_gather(win_v.at[slot], [lanes, lane_i])
            v_hi = plsc.load_gather(win_v.at[slot], [rows_hi, lane_i])
            part[pl.ds(pl.multiple_of(r * L, L), L)] = (
                u_lo * v_lo + u_hi * v_hi)

        for r in range(NB - 1):
            iu, ii = scal2(r)
            fetch(r, sems[r], iu, ii)

        def body(b, carry):
            for s in range(NB):
                r = NB * b + s
                nslot = (s + NB - 1) % NB
                iu_n, ii_n = scal2(r + NB - 1)

                @pl.when(r + NB - 1 < BPW)
                def _():
                    fetch(nslot, sems[nslot], iu_n, ii_n)

                drain(s, sems[s])
                iu, ii = scal2(r)
                compute(r, s, iu, ii)
            return carry

        lax.fori_loop(0, BPW // NB, body, 0)

        def red_body(g, carry):
            row0 = pl.multiple_of(g * L, L)
            flat0 = row0 * L + lax.iota(jnp.int32, L) * L
            acc = jnp.zeros((L,), jnp.float32)
            for j in range(L):
                acc = acc + plsc.load_gather(part, [flat0 + j])
            out_v[pl.ds(row0, L)] = acc
            return carry

        lax.fori_loop(0, BPW // L, red_body, 0)

        base = pl.multiple_of(wid * BPW, BPW)
        pltpu.sync_copy(out_v, out_hbm.at[pl.ds(base, BPW)])

    return k(data2, ut, it)


def kernel(data, user_factors, item_factors):
    data2 = data.astype(jnp.int32).reshape(NW, 2 * BPW)
    return _sc_dot(data2, user_factors.T, item_factors.T)


# speedup vs baseline: 5.1485x; 1.0007x over previous
import functools

import jax
import jax.numpy as jnp
from jax import lax
from jax.experimental import pallas as pl
from jax.experimental.pallas import tpu as pltpu
from jax.experimental.pallas import tpu_sc as plsc

B = 16384
D = 32
LANES = 128

_info = plsc.get_sparse_core_info()
NC = _info.num_cores
NS = _info.num_subcores
L = _info.num_lanes
NW = NC * NS
BPW = B // NW
NB = 8


def _sc_dot(data2, ut, it):
    mesh = plsc.VectorSubcoreMesh(core_axis_name="c", subcore_axis_name="s")

    @functools.partial(
        pl.kernel,
        mesh=mesh,
        compiler_params=pltpu.CompilerParams(
            needs_layout_passes=False, use_tc_tiling_on_sc=True),
        out_type=jax.ShapeDtypeStruct((B,), jnp.float32),
        scratch_types=[
            pltpu.VMEM((2 * BPW,), jnp.int32),
            pltpu.VMEM((NB, D, LANES), jnp.float32),
            pltpu.VMEM((NB, D, LANES), jnp.float32),
            pltpu.VMEM((BPW * L,), jnp.float32),
            pltpu.VMEM((BPW,), jnp.float32),
            [pltpu.SemaphoreType.DMA] * NB,
        ],
    )
    def k(d2_hbm, ut_hbm, it_hbm, out_hbm,
          idx_vm, win_u, win_v, part, out_v, sems):
        wid = lax.axis_index("s") * NC + lax.axis_index("c")

        pltpu.sync_copy(d2_hbm.at[wid], idx_vm)

        lanes = lax.iota(jnp.int32, L)

        def scal2(r):
            pos = jnp.minimum(2 * r, 2 * BPW - 2)
            base = pl.multiple_of((pos // L) * L, L)
            vec = idx_vm[pl.ds(base, L)]
            off = pos - base
            zero = jnp.zeros((L,), jnp.int32)
            iu = jnp.sum(jnp.where(lanes == jnp.full((L,), off), vec, zero))
            ii = jnp.sum(
                jnp.where(lanes == jnp.full((L,), off + 1), vec, zero))
            return iu, ii

        def fetch(slot, sem, iu, ii):
            cu = pl.multiple_of((iu // LANES) * LANES, LANES)
            ci = pl.multiple_of((ii // LANES) * LANES, LANES)
            for q in range(4):
                sl = pl.ds(8 * q, 8)
                pltpu.async_copy(
                    ut_hbm.at[sl, pl.ds(cu, LANES)],
                    win_u.at[slot].at[sl], sem)
                pltpu.async_copy(
                    it_hbm.at[sl, pl.ds(ci, LANES)],
                    win_v.at[slot].at[sl], sem)

        def drain(slot, sem):
            for q in range(4):
                sl = pl.ds(8 * q, 8)
                pltpu.make_async_copy(
                    ut_hbm.at[sl, pl.ds(0, LANES)],
                    win_u.at[slot].at[sl], sem).wait()
                pltpu.make_async_copy(
                    it_hbm.at[sl, pl.ds(0, LANES)],
                    win_v.at[slot].at[sl], sem).wait()

        rows_hi = lanes + L

        def compute(r, slot, iu, ii):
            lane_u = jnp.full((L,), iu % LANES, jnp.int32)
            lane_i = jnp.full((L,), ii % LANES, jnp.int32)
            u_lo = plsc.load_gather(win_u.at[slot], [lanes, lane_u])
            u_hi = plsc.load_gather(win_u.at[slot], [rows_hi, lane_u])
            v_lo = plsc.load_gather(win_v.at[slot], [lanes, lane_i])
            v_hi = plsc.load_gather(win_v.at[slot], [rows_hi, lane_i])
            part[pl.ds(pl.multiple_of(r * L, L), L)] = (
                u_lo * v_lo + u_hi * v_hi)

        for r in range(NB - 1):
            iu, ii = scal2(r)
            fetch(r, sems[r], iu, ii)

        def body(b, carry):
            for s in range(NB):
                r = NB * b + s
                nslot = (s + NB - 1) % NB
                iu_n, ii_n = scal2(r + NB - 1)

                @pl.when(r + NB - 1 < BPW)
                def _():
                    fetch(nslot, sems[nslot], iu_n, ii_n)

                drain(s, sems[s])
                iu, ii = scal2(r)
                compute(r, s, iu, ii)
            return carry

        lax.fori_loop(0, BPW // NB, body, 0)

        def red_body(g, carry):
            row0 = pl.multiple_of(g * L, L)
            flat0 = row0 * L + lax.iota(jnp.int32, L) * L
            acc = jnp.zeros((L,), jnp.float32)
            for j in range(L):
                acc = acc + plsc.load_gather(part, [flat0 + j])
            out_v[pl.ds(row0, L)] = acc
            return carry

        lax.fori_loop(0, BPW // L, red_body, 0)

        base = pl.multiple_of(wid * BPW, BPW)
        pltpu.sync_copy(out_v, out_hbm.at[pl.ds(base, BPW)])

    return k(data2, ut, it)


def kernel(data, user_factors, item_factors):
    data2 = data.astype(jnp.int32).reshape(NW, 2 * BPW)
    return _sc_dot(data2, user_factors.T, item_factors.T)
